# baseline (speedup 1.0000x reference)
import functools

import jax
import jax.numpy as jnp
from jax.experimental import pallas as pl

_N_E = 8192
_E_DIM = 256
_NUM_Q = 4
_BETA = 0.25
_BLK = 256


def _stage_body(last, r_ref, a_ref, cb_ref, cbt_ref, x_ref, idx_ref, rout_ref,
                loss_ref, xq_ref=None):
    rblk = r_ref[...]
    cbt = cbt_ref[...]
    s = jnp.dot(rblk, cbt, preferred_element_type=jnp.float32)
    a = a_ref[...]
    n = jnp.sum(cbt * cbt, axis=0, keepdims=True)
    d = (a - 2.0 * s) + n
    m = jnp.min(d, axis=1, keepdims=True)
    iota = jax.lax.broadcasted_iota(jnp.int32, (_BLK, _N_E), 1)
    idx = jnp.min(jnp.where(d == m, iota, _N_E), axis=1)
    idx_ref[...] = idx.reshape(1, 1, _BLK)
    oh = (iota == idx[:, None]).astype(jnp.float32)
    zq = jax.lax.dot_general(oh, cb_ref[...], (((1,), (0,)), ((), ())),
                             precision=jax.lax.Precision.HIGHEST,
                             preferred_element_type=jnp.float32)
    t = zq - rblk
    zst = rblk + t
    rnew = rblk - zst
    rout_ref[...] = rnew
    loss_ref[...] = jnp.broadcast_to(jnp.sum(t * t), (1, 1, 128))
    if last:
        xq_ref[...] = x_ref[...] - rnew


def _run_stage(r, a, cb, cbt, x, last):
    n_tok = r.shape[0]
    nblk = n_tok // _BLK
    out_shapes = [
        jax.ShapeDtypeStruct((nblk, 1, _BLK), jnp.int32),
        jax.ShapeDtypeStruct((n_tok, _E_DIM), jnp.float32),
        jax.ShapeDtypeStruct((nblk, 1, 128), jnp.float32),
    ]
    out_specs = [
        pl.BlockSpec((1, 1, _BLK), lambda i: (i, 0, 0)),
        pl.BlockSpec((_BLK, _E_DIM), lambda i: (i, 0)),
        pl.BlockSpec((1, 1, 128), lambda i: (i, 0, 0)),
    ]
    in_specs = [
        pl.BlockSpec((_BLK, _E_DIM), lambda i: (i, 0)),
        pl.BlockSpec((_BLK, 1), lambda i: (i, 0)),
        pl.BlockSpec((_N_E, _E_DIM), lambda i: (0, 0)),
        pl.BlockSpec((_E_DIM, _N_E), lambda i: (0, 0)),
        pl.BlockSpec((_BLK, _E_DIM), lambda i: (i, 0)),
    ]
    if last:
        out_shapes.append(jax.ShapeDtypeStruct((n_tok, _E_DIM), jnp.float32))
        out_specs.append(pl.BlockSpec((_BLK, _E_DIM), lambda i: (i, 0)))
    return pl.pallas_call(
        functools.partial(_stage_body, last),
        grid=(nblk,),
        in_specs=in_specs,
        out_specs=out_specs,
        out_shape=out_shapes,
    )(r, a, cb, cbt, x)


def kernel(x, codebook_0, codebook_1, codebook_2, codebook_3):
    codebooks = [codebook_0, codebook_1, codebook_2, codebook_3]
    batch, tokens, dim = x.shape
    xflat = x.reshape(-1, dim)
    r = xflat
    idxs = []
    losses = []
    xq = None
    for i, cb in enumerate(codebooks):
        last = i == _NUM_Q - 1
        a = jnp.sum(r ** 2, axis=1, keepdims=True)
        outs = _run_stage(r, a, cb, cb.T, xflat, last)
        if last:
            idx, r, lpart, xq = outs
        else:
            idx, r, lpart = outs
        idxs.append(idx.reshape(batch, tokens))
        m = jnp.sum(lpart[:, 0, 0]) / (xflat.shape[0] * dim)
        losses.append(m + _BETA * m)
    mean_losses = jnp.stack(losses).mean()
    all_indices = jnp.stack(idxs, axis=-1)
    return (xq.reshape(x.shape), mean_losses, all_indices)


# speedup vs baseline: 1.3929x; 1.3929x over previous
import functools

import jax
import jax.numpy as jnp
from jax.experimental import pallas as pl

_N_E = 8192
_E_DIM = 256
_NUM_Q = 4
_BETA = 0.25
_BLK = 256


def _stage_body(last, r_ref, a_ref, cb_ref, cbt_ref, x_ref, idx_ref, rout_ref,
                loss_ref, xq_ref=None):
    rblk = r_ref[...]
    cbt = cbt_ref[...]
    s = jnp.dot(rblk, cbt, preferred_element_type=jnp.float32)
    a = a_ref[...]
    n = jnp.sum(cbt * cbt, axis=0, keepdims=True)
    d = (a - 2.0 * s) + n
    m = jnp.min(d, axis=1, keepdims=True)
    iota = jax.lax.broadcasted_iota(jnp.int32, (_BLK, _N_E), 1)
    idx = jnp.min(jnp.where(d == m, iota, _N_E), axis=1)
    idx_ref[...] = idx.reshape(1, 1, _BLK)
    oh = (iota == idx[:, None]).astype(jnp.bfloat16)
    parts = [
        jnp.dot(oh, p_ref[...], preferred_element_type=jnp.float32)
        for p_ref in cb_ref
    ]
    zq = (parts[0] + parts[1]) + parts[2]
    t = zq - rblk
    zst = rblk + t
    rnew = rblk - zst
    rout_ref[...] = rnew
    loss_ref[...] = jnp.broadcast_to(jnp.sum(t * t), (1, 1, 128))
    if last:
        xq_ref[...] = x_ref[...] - rnew


def _stage_body_wrap(last, r_ref, a_ref, hi_ref, mid_ref, lo_ref, cbt_ref,
                     x_ref, *out_refs):
    _stage_body(last, r_ref, a_ref, (hi_ref, mid_ref, lo_ref), cbt_ref,
                x_ref, *out_refs)


def _run_stage(r, a, cb_parts, cbt, x, last):
    n_tok = r.shape[0]
    nblk = n_tok // _BLK
    out_shapes = [
        jax.ShapeDtypeStruct((nblk, 1, _BLK), jnp.int32),
        jax.ShapeDtypeStruct((n_tok, _E_DIM), jnp.float32),
        jax.ShapeDtypeStruct((nblk, 1, 128), jnp.float32),
    ]
    out_specs = [
        pl.BlockSpec((1, 1, _BLK), lambda i: (i, 0, 0)),
        pl.BlockSpec((_BLK, _E_DIM), lambda i: (i, 0)),
        pl.BlockSpec((1, 1, 128), lambda i: (i, 0, 0)),
    ]
    in_specs = [
        pl.BlockSpec((_BLK, _E_DIM), lambda i: (i, 0)),
        pl.BlockSpec((_BLK, 1), lambda i: (i, 0)),
        pl.BlockSpec((_N_E, _E_DIM), lambda i: (0, 0)),
        pl.BlockSpec((_N_E, _E_DIM), lambda i: (0, 0)),
        pl.BlockSpec((_N_E, _E_DIM), lambda i: (0, 0)),
        pl.BlockSpec((_E_DIM, _N_E), lambda i: (0, 0)),
        pl.BlockSpec((_BLK, _E_DIM), lambda i: (i, 0)),
    ]
    if last:
        out_shapes.append(jax.ShapeDtypeStruct((n_tok, _E_DIM), jnp.float32))
        out_specs.append(pl.BlockSpec((_BLK, _E_DIM), lambda i: (i, 0)))
    return pl.pallas_call(
        functools.partial(_stage_body_wrap, last),
        grid=(nblk,),
        in_specs=in_specs,
        out_specs=out_specs,
        out_shape=out_shapes,
    )(r, a, cb_parts[0], cb_parts[1], cb_parts[2], cbt, x)


def kernel(x, codebook_0, codebook_1, codebook_2, codebook_3):
    codebooks = [codebook_0, codebook_1, codebook_2, codebook_3]
    batch, tokens, dim = x.shape
    xflat = x.reshape(-1, dim)
    r = xflat
    idxs = []
    losses = []
    xq = None
    for i, cb in enumerate(codebooks):
        last = i == _NUM_Q - 1
        hi = cb.astype(jnp.bfloat16)
        mid = (cb - hi.astype(jnp.float32)).astype(jnp.bfloat16)
        lo = ((cb - hi.astype(jnp.float32)) - mid.astype(jnp.float32)
              ).astype(jnp.bfloat16)
        a = jnp.sum(r ** 2, axis=1, keepdims=True)
        outs = _run_stage(r, a, (hi, mid, lo), cb.T, xflat, last)
        if last:
            idx, r, lpart, xq = outs
        else:
            idx, r, lpart = outs
        idxs.append(idx.reshape(batch, tokens))
        m = jnp.sum(lpart[:, 0, 0]) / (xflat.shape[0] * dim)
        losses.append(m + _BETA * m)
    mean_losses = jnp.stack(losses).mean()
    all_indices = jnp.stack(idxs, axis=-1)
    return (xq.reshape(x.shape), mean_losses, all_indices)
